# baseline (speedup 1.0000x reference)
import functools

import jax
import jax.numpy as jnp
from jax import lax
from jax.experimental import pallas as pl
from jax.experimental.pallas import tpu as pltpu
from jax.experimental.pallas import tpu_sc as plsc

_E = 64
_NW = 32
_TPW = 256
_L = 16
_G = _TPW // _L


def _gate_body(x_ref, wt_ref, p_ref, me_ref, acc_ref):
    i = pl.program_id(0)

    @pl.when(i == 0)
    def _init():
        acc_ref[...] = jnp.zeros_like(acc_ref)

    logits = jnp.dot(x_ref[...], wt_ref[...],
                     preferred_element_type=jnp.float32)
    m = jnp.max(logits, axis=-1, keepdims=True)
    ex = jnp.exp(logits - m)
    s = jnp.sum(ex, axis=-1, keepdims=True)
    p = ex / s
    p_ref[0] = p.T
    acc_ref[...] += jnp.sum(p, axis=0, keepdims=True)

    @pl.when(i == pl.num_programs(0) - 1)
    def _fini():
        me_ref[...] = acc_ref[...]


def _tc_gate(xf, wt):
    d = xf.shape[1]
    return pl.pallas_call(
        _gate_body,
        grid=(_NW,),
        in_specs=[
            pl.BlockSpec((_TPW, d), lambda i: (i, 0)),
            pl.BlockSpec((d, _E), lambda i: (0, 0)),
        ],
        out_specs=[
            pl.BlockSpec((1, _E, _TPW), lambda i: (i, 0, 0)),
            pl.BlockSpec((1, _E), lambda i: (0, 0)),
        ],
        out_shape=[
            jax.ShapeDtypeStruct((_NW, _E, _TPW), jnp.float32),
            jax.ShapeDtypeStruct((1, _E), jnp.float32),
        ],
        scratch_shapes=[pltpu.VMEM((1, _E), jnp.float32)],
    )(xf, wt)


def _route_body(p_hbm, imp_hbm, ind_hbm, ce_hbm,
                lg, impv, indv, cebuf, sem):
    wid = lax.axis_index("s") * 2 + lax.axis_index("c")
    pltpu.sync_copy(p_hbm.at[wid], lg)

    zeros = jnp.zeros((_L,), jnp.float32)

    def _zero_tiles(j, c):
        for g in range(_G):
            impv[j, pl.ds(g * _L, _L)] = zeros
            indv[j, pl.ds(g * _L, _L)] = zeros
        cebuf[j, :] = zeros
        return c

    lax.fori_loop(0, _E, _zero_tiles, 0)

    lane = lax.iota(jnp.int32, _L)
    ones = jnp.ones((_L,), jnp.float32)
    izeros = jnp.zeros((_L,), jnp.int32)

    def _group_pair(j, c):
        g0 = j * 2
        g1 = g0 + 1
        col0 = g0 * _L + lane
        col1 = g1 * _L + lane

        def _estep(eo, carry):
            m1a, m2a, a1a, a2a, m1b, m2b, a1b, a2b = carry
            for eu in range(8):
                e = eo * 8 + eu
                va = lg[e, pl.ds(g0 * _L, _L)]
                vb = lg[e, pl.ds(g1 * _L, _L)]
                gt1a = va > m1a
                gt2a = va > m2a
                m2a = jnp.where(gt1a, m1a, jnp.where(gt2a, va, m2a))
                a2a = jnp.where(gt1a, a1a, jnp.where(gt2a, e, a2a))
                m1a = jnp.where(gt1a, va, m1a)
                a1a = jnp.where(gt1a, e, a1a)
                gt1b = vb > m1b
                gt2b = vb > m2b
                m2b = jnp.where(gt1b, m1b, jnp.where(gt2b, vb, m2b))
                a2b = jnp.where(gt1b, a1b, jnp.where(gt2b, e, a2b))
                m1b = jnp.where(gt1b, vb, m1b)
                a1b = jnp.where(gt1b, e, a1b)
            return (m1a, m2a, a1a, a2a, m1b, m2b, a1b, a2b)

        neg = jnp.full((_L,), -1.0, jnp.float32)
        m1a, m2a, a1a, a2a, m1b, m2b, a1b, a2b = lax.fori_loop(
            0, _E // 8, _estep,
            (neg, neg, izeros, izeros, neg, neg, izeros, izeros))

        plsc.store_scatter(impv, [a1a, col0], m1a)
        plsc.store_scatter(impv, [a2a, col0], m2a)
        plsc.store_scatter(indv, [a1a, col0], ones)
        plsc.store_scatter(indv, [a2a, col0], ones)
        plsc.store_scatter(impv, [a1b, col1], m1b)
        plsc.store_scatter(impv, [a2b, col1], m2b)
        plsc.store_scatter(indv, [a1b, col1], ones)
        plsc.store_scatter(indv, [a2b, col1], ones)
        plsc.addupdate_scatter(cebuf, [a1a, lane], ones)
        plsc.addupdate_scatter(cebuf, [a1b, lane], ones)
        return c

    lax.fori_loop(0, _G // 2, _group_pair, 0)

    c1 = pltpu.async_copy(impv, imp_hbm.at[:, wid], sem)
    c2 = pltpu.async_copy(indv, ind_hbm.at[:, wid], sem)
    c1.wait()
    c2.wait()
    pltpu.sync_copy(cebuf, ce_hbm.at[wid])


_SC_MESH = plsc.VectorSubcoreMesh(
    core_axis_name="c", subcore_axis_name="s", num_cores=2, num_subcores=16)

_sc_route = pl.kernel(
    _route_body,
    out_type=[
        jax.ShapeDtypeStruct((_E, _NW, _TPW), jnp.float32),
        jax.ShapeDtypeStruct((_E, _NW, _TPW), jnp.float32),
        jax.ShapeDtypeStruct((_NW, _E, _L), jnp.float32),
    ],
    mesh=_SC_MESH,
    scratch_types=[
        pltpu.VMEM((_E, _TPW), jnp.float32),
        pltpu.VMEM((_E, _TPW), jnp.float32),
        pltpu.VMEM((_E, _TPW), jnp.float32),
        pltpu.VMEM((_E, _L), jnp.float32),
        pltpu.SemaphoreType.DMA,
    ],
    compiler_params=pltpu.CompilerParams(needs_layout_passes=False),
)


def kernel(x, W):
    B, S, D = x.shape
    n = B * S
    xf = x.reshape(n, D)
    p3, me_sum = _tc_gate(xf, W.T)
    imp3, ind3, ce_p = _sc_route(p3)
    imp = imp3.reshape(_E, B, S, 1)
    ind = ind3.reshape(_E, B, S, 1)
    ce = jnp.sum(ce_p, axis=(0, 2))
    loss = jnp.sum(me_sum[0] * ce) * (_E / float(n * n))
    return imp, ind, loss


# speedup vs baseline: 1.0139x; 1.0139x over previous
import functools

import jax
import jax.numpy as jnp
from jax import lax
from jax.experimental import pallas as pl
from jax.experimental.pallas import tpu as pltpu
from jax.experimental.pallas import tpu_sc as plsc

_E = 64
_NW = 32
_TPW = 256
_L = 16
_G = _TPW // _L


def _gate_body(x_ref, wt_ref, p_ref, me_ref, acc_ref):
    i = pl.program_id(0)

    @pl.when(i == 0)
    def _init():
        acc_ref[...] = jnp.zeros_like(acc_ref)

    logits = jnp.dot(x_ref[...], wt_ref[...],
                     preferred_element_type=jnp.float32)
    m = jnp.max(logits, axis=-1, keepdims=True)
    ex = jnp.exp(logits - m)
    s = jnp.sum(ex, axis=-1, keepdims=True)
    p = ex / s
    p_ref[0] = p.T
    acc_ref[...] += jnp.sum(p, axis=0, keepdims=True)

    @pl.when(i == pl.num_programs(0) - 1)
    def _fini():
        me_ref[...] = acc_ref[...]


def _tc_gate(xf, wt):
    d = xf.shape[1]
    return pl.pallas_call(
        _gate_body,
        grid=(_NW,),
        in_specs=[
            pl.BlockSpec((_TPW, d), lambda i: (i, 0)),
            pl.BlockSpec((d, _E), lambda i: (0, 0)),
        ],
        out_specs=[
            pl.BlockSpec((1, _E, _TPW), lambda i: (i, 0, 0)),
            pl.BlockSpec((1, _E), lambda i: (0, 0)),
        ],
        out_shape=[
            jax.ShapeDtypeStruct((_NW, _E, _TPW), jnp.float32),
            jax.ShapeDtypeStruct((1, _E), jnp.float32),
        ],
        scratch_shapes=[pltpu.VMEM((1, _E), jnp.float32)],
    )(xf, wt)


def _route_body(p_hbm, imp_hbm, ind_hbm, ce_hbm,
                lg, impv, indv, cebuf, sem):
    wid = lax.axis_index("s") * 2 + lax.axis_index("c")
    cin = pltpu.async_copy(p_hbm.at[wid], lg, sem)

    zeros = jnp.zeros((_L,), jnp.float32)

    def _zero_tiles(j, c):
        for g in range(_G):
            impv[j, pl.ds(g * _L, _L)] = zeros
            indv[j, pl.ds(g * _L, _L)] = zeros
        cebuf[j, :] = zeros
        return c

    lax.fori_loop(0, _E, _zero_tiles, 0)
    cin.wait()

    lane = lax.iota(jnp.int32, _L)
    ones = jnp.ones((_L,), jnp.float32)
    izeros = jnp.zeros((_L,), jnp.int32)

    def _group_pair(j, c):
        g0 = j * 2
        g1 = g0 + 1
        col0 = g0 * _L + lane
        col1 = g1 * _L + lane

        def _estep(eo, carry):
            m1a, m2a, a1a, a2a, m1b, m2b, a1b, a2b = carry
            for eu in range(8):
                e = eo * 8 + eu
                va = lg[e, pl.ds(g0 * _L, _L)]
                vb = lg[e, pl.ds(g1 * _L, _L)]
                gt1a = va > m1a
                gt2a = va > m2a
                m2a = jnp.where(gt1a, m1a, jnp.where(gt2a, va, m2a))
                a2a = jnp.where(gt1a, a1a, jnp.where(gt2a, e, a2a))
                m1a = jnp.where(gt1a, va, m1a)
                a1a = jnp.where(gt1a, e, a1a)
                gt1b = vb > m1b
                gt2b = vb > m2b
                m2b = jnp.where(gt1b, m1b, jnp.where(gt2b, vb, m2b))
                a2b = jnp.where(gt1b, a1b, jnp.where(gt2b, e, a2b))
                m1b = jnp.where(gt1b, vb, m1b)
                a1b = jnp.where(gt1b, e, a1b)
            return (m1a, m2a, a1a, a2a, m1b, m2b, a1b, a2b)

        neg = jnp.full((_L,), -1.0, jnp.float32)
        m1a, m2a, a1a, a2a, m1b, m2b, a1b, a2b = lax.fori_loop(
            0, _E // 8, _estep,
            (neg, neg, izeros, izeros, neg, neg, izeros, izeros))

        plsc.store_scatter(impv, [a1a, col0], m1a)
        plsc.store_scatter(impv, [a2a, col0], m2a)
        plsc.store_scatter(indv, [a1a, col0], ones)
        plsc.store_scatter(indv, [a2a, col0], ones)
        plsc.store_scatter(impv, [a1b, col1], m1b)
        plsc.store_scatter(impv, [a2b, col1], m2b)
        plsc.store_scatter(indv, [a1b, col1], ones)
        plsc.store_scatter(indv, [a2b, col1], ones)
        plsc.addupdate_scatter(cebuf, [a1a, lane], ones)
        plsc.addupdate_scatter(cebuf, [a1b, lane], ones)
        return c

    lax.fori_loop(0, _G // 2, _group_pair, 0)

    c1 = pltpu.async_copy(impv, imp_hbm.at[:, wid], sem)
    c2 = pltpu.async_copy(indv, ind_hbm.at[:, wid], sem)
    c1.wait()
    c2.wait()
    pltpu.sync_copy(cebuf, ce_hbm.at[wid])


_SC_MESH = plsc.VectorSubcoreMesh(
    core_axis_name="c", subcore_axis_name="s", num_cores=2, num_subcores=16)

_sc_route = pl.kernel(
    _route_body,
    out_type=[
        jax.ShapeDtypeStruct((_E, _NW, _TPW), jnp.float32),
        jax.ShapeDtypeStruct((_E, _NW, _TPW), jnp.float32),
        jax.ShapeDtypeStruct((_NW, _E, _L), jnp.float32),
    ],
    mesh=_SC_MESH,
    scratch_types=[
        pltpu.VMEM((_E, _TPW), jnp.float32),
        pltpu.VMEM((_E, _TPW), jnp.float32),
        pltpu.VMEM((_E, _TPW), jnp.float32),
        pltpu.VMEM((_E, _L), jnp.float32),
        pltpu.SemaphoreType.DMA,
    ],
    compiler_params=pltpu.CompilerParams(needs_layout_passes=False),
)


def kernel(x, W):
    B, S, D = x.shape
    n = B * S
    xf = x.reshape(n, D)
    p3, me_sum = _tc_gate(xf, W.T)
    imp3, ind3, ce_p = _sc_route(p3)
    imp = imp3.reshape(_E, B, S, 1)
    ind = ind3.reshape(_E, B, S, 1)
    ce = jnp.sum(ce_p, axis=(0, 2))
    loss = jnp.sum(me_sum[0] * ce) * (_E / float(n * n))
    return imp, ind, loss


# speedup vs baseline: 1.0145x; 1.0005x over previous
import jax
import jax.numpy as jnp
from jax import lax
from jax.experimental import pallas as pl
from jax.experimental.pallas import tpu as pltpu
from jax.experimental.pallas import tpu_sc as plsc

_E = 64
_NW = 32
_TPW = 256
_L = 16
_G = _TPW // _L


def _gate_body(x_ref, wt_ref, p_ref, me_ref, acc_ref):
    i = pl.program_id(0)

    @pl.when(i == 0)
    def _init():
        acc_ref[...] = jnp.zeros_like(acc_ref)

    logits = jnp.dot(x_ref[...], wt_ref[...],
                     preferred_element_type=jnp.float32)
    m = jnp.max(logits, axis=-1, keepdims=True)
    ex = jnp.exp(logits - m)
    s = jnp.sum(ex, axis=-1, keepdims=True)
    p = ex / s
    p_ref[0] = p.T
    acc_ref[...] += jnp.sum(p, axis=0, keepdims=True)

    @pl.when(i == pl.num_programs(0) - 1)
    def _fini():
        me_ref[...] = acc_ref[...]


def _tc_gate(xf, wt):
    d = xf.shape[1]
    return pl.pallas_call(
        _gate_body,
        grid=(_NW,),
        in_specs=[
            pl.BlockSpec((_TPW, d), lambda i: (i, 0)),
            pl.BlockSpec((d, _E), lambda i: (0, 0)),
        ],
        out_specs=[
            pl.BlockSpec((1, _E, _TPW), lambda i: (i, 0, 0)),
            pl.BlockSpec((1, _E), lambda i: (0, 0)),
        ],
        out_shape=[
            jax.ShapeDtypeStruct((_NW, _E, _TPW), jnp.float32),
            jax.ShapeDtypeStruct((1, _E), jnp.float32),
        ],
        scratch_shapes=[pltpu.VMEM((1, _E), jnp.float32)],
    )(xf, wt)


def _route_body(p_hbm, imp_hbm, ind_hbm, ce_hbm,
                lg, impv, indv, cebuf, sem):
    wid = lax.axis_index("s") * 2 + lax.axis_index("c")
    cin = pltpu.async_copy(p_hbm.at[wid], lg, sem)

    zeros = jnp.zeros((_L,), jnp.float32)

    def _zero_tiles(j, c):
        for g in range(_G):
            impv[j, pl.ds(g * _L, _L)] = zeros
            indv[j, pl.ds(g * _L, _L)] = zeros
        cebuf[j, :] = zeros
        return c

    lax.fori_loop(0, _E, _zero_tiles, 0)
    cin.wait()

    lane = lax.iota(jnp.int32, _L)
    ones = jnp.ones((_L,), jnp.float32)
    izeros = jnp.zeros((_L,), jnp.int32)

    def _group_pair(j, c):
        g0 = j * 2
        g1 = g0 + 1
        col0 = g0 * _L + lane
        col1 = g1 * _L + lane

        def _estep(eo, carry):
            m1a, m2a, a1a, a2a, m1b, m2b, a1b, a2b = carry
            for eu in range(8):
                e = eo * 8 + eu
                va = lg[e, pl.ds(g0 * _L, _L)]
                vb = lg[e, pl.ds(g1 * _L, _L)]
                gt1a = va > m1a
                gt2a = va > m2a
                m2a = jnp.where(gt1a, m1a, jnp.where(gt2a, va, m2a))
                a2a = jnp.where(gt1a, a1a, jnp.where(gt2a, e, a2a))
                m1a = jnp.where(gt1a, va, m1a)
                a1a = jnp.where(gt1a, e, a1a)
                gt1b = vb > m1b
                gt2b = vb > m2b
                m2b = jnp.where(gt1b, m1b, jnp.where(gt2b, vb, m2b))
                a2b = jnp.where(gt1b, a1b, jnp.where(gt2b, e, a2b))
                m1b = jnp.where(gt1b, vb, m1b)
                a1b = jnp.where(gt1b, e, a1b)
            return (m1a, m2a, a1a, a2a, m1b, m2b, a1b, a2b)

        neg = jnp.full((_L,), -1.0, jnp.float32)
        m1a, m2a, a1a, a2a, m1b, m2b, a1b, a2b = lax.fori_loop(
            0, _E // 8, _estep,
            (neg, neg, izeros, izeros, neg, neg, izeros, izeros))

        plsc.store_scatter(impv, [a1a, col0], m1a)
        plsc.store_scatter(impv, [a2a, col0], m2a)
        plsc.store_scatter(indv, [a1a, col0], ones)
        plsc.store_scatter(indv, [a2a, col0], ones)
        plsc.store_scatter(impv, [a1b, col1], m1b)
        plsc.store_scatter(impv, [a2b, col1], m2b)
        plsc.store_scatter(indv, [a1b, col1], ones)
        plsc.store_scatter(indv, [a2b, col1], ones)
        plsc.addupdate_scatter(cebuf, [a1a, lane], ones)
        plsc.addupdate_scatter(cebuf, [a1b, lane], ones)
        return c

    lax.fori_loop(0, _G // 2, _group_pair, 0)

    c1 = pltpu.async_copy(impv, imp_hbm.at[:, wid], sem)
    c2 = pltpu.async_copy(indv, ind_hbm.at[:, wid], sem)
    c1.wait()
    c2.wait()
    pltpu.sync_copy(cebuf, ce_hbm.at[wid])


_SC_MESH = plsc.VectorSubcoreMesh(
    core_axis_name="c", subcore_axis_name="s", num_cores=2, num_subcores=16)

_sc_route = pl.kernel(
    _route_body,
    out_type=[
        jax.ShapeDtypeStruct((_E, _NW, _TPW), jnp.float32),
        jax.ShapeDtypeStruct((_E, _NW, _TPW), jnp.float32),
        jax.ShapeDtypeStruct((_NW, _E, _L), jnp.float32),
    ],
    mesh=_SC_MESH,
    scratch_types=[
        pltpu.VMEM((_E, _TPW), jnp.float32),
        pltpu.VMEM((_E, _TPW), jnp.float32),
        pltpu.VMEM((_E, _TPW), jnp.float32),
        pltpu.VMEM((_E, _L), jnp.float32),
        pltpu.SemaphoreType.DMA,
    ],
    compiler_params=pltpu.CompilerParams(needs_layout_passes=False),
)


def kernel(x, W):
    B, S, D = x.shape
    n = B * S
    xf = x.reshape(n, D)
    p3, me_sum = _tc_gate(xf, W.T)
    imp3, ind3, ce_p = _sc_route(p3)
    imp = imp3.reshape(_E, B, S, 1)
    ind = ind3.reshape(_E, B, S, 1)
    ce = jnp.sum(ce_p, axis=(0, 2))
    loss = jnp.sum(me_sum[0] * ce) * (_E / float(n * n))
    return imp, ind, loss


# speedup vs baseline: 1.1651x; 1.1485x over previous
import jax
import jax.numpy as jnp
from jax import lax
from jax.experimental import pallas as pl
from jax.experimental.pallas import tpu as pltpu
from jax.experimental.pallas import tpu_sc as plsc

_E = 64
_NW = 32
_TPW = 256
_L = 16
_G = _TPW // _L


_TB = 512


def _gate_body(x_ref, wt_ref, p_ref, me_ref, acc_ref):
    i = pl.program_id(0)

    @pl.when(i == 0)
    def _init():
        acc_ref[...] = jnp.zeros_like(acc_ref)

    logits = jnp.dot(x_ref[...], wt_ref[...],
                     preferred_element_type=jnp.float32)
    m = jnp.max(logits, axis=-1, keepdims=True)
    ex = jnp.exp(logits - m)
    s = jnp.sum(ex, axis=-1, keepdims=True)
    p = ex / s
    p_ref[...] = p.T
    acc_ref[...] += jnp.sum(p, axis=0, keepdims=True)

    @pl.when(i == pl.num_programs(0) - 1)
    def _fini():
        me_ref[...] = acc_ref[...]


def _tc_gate(xf, wt):
    n, d = xf.shape
    return pl.pallas_call(
        _gate_body,
        grid=(n // _TB,),
        in_specs=[
            pl.BlockSpec((_TB, d), lambda i: (i, 0)),
            pl.BlockSpec((d, _E), lambda i: (0, 0)),
        ],
        out_specs=[
            pl.BlockSpec((_E, _TB), lambda i: (0, i)),
            pl.BlockSpec((1, _E), lambda i: (0, 0)),
        ],
        out_shape=[
            jax.ShapeDtypeStruct((_E, n), jnp.float32),
            jax.ShapeDtypeStruct((1, _E), jnp.float32),
        ],
        scratch_shapes=[pltpu.VMEM((1, _E), jnp.float32)],
    )(xf, wt)


def _route_body(p_hbm, imp_hbm, ind_hbm, ce_hbm,
                lg, impv, indv, cebuf, sem):
    wid = lax.axis_index("s") * 2 + lax.axis_index("c")
    base = wid * _TPW
    cin = pltpu.async_copy(p_hbm.at[:, pl.ds(base, _TPW)], lg, sem)

    zeros = jnp.zeros((_L,), jnp.float32)

    def _zero_tiles(j, c):
        for g in range(_G):
            impv[j, pl.ds(g * _L, _L)] = zeros
            indv[j, pl.ds(g * _L, _L)] = zeros
        cebuf[j, :] = zeros
        return c

    lax.fori_loop(0, _E, _zero_tiles, 0)
    cin.wait()

    lane = lax.iota(jnp.int32, _L)
    ones = jnp.ones((_L,), jnp.float32)
    izeros = jnp.zeros((_L,), jnp.int32)

    def _group_pair(j, c):
        g0 = j * 2
        g1 = g0 + 1
        col0 = g0 * _L + lane
        col1 = g1 * _L + lane

        def _estep(eo, carry):
            m1a, m2a, a1a, a2a, m1b, m2b, a1b, a2b = carry
            for eu in range(8):
                e = eo * 8 + eu
                va = lg[e, pl.ds(g0 * _L, _L)]
                vb = lg[e, pl.ds(g1 * _L, _L)]
                gt1a = va > m1a
                gt2a = va > m2a
                m2a = jnp.where(gt1a, m1a, jnp.where(gt2a, va, m2a))
                a2a = jnp.where(gt1a, a1a, jnp.where(gt2a, e, a2a))
                m1a = jnp.where(gt1a, va, m1a)
                a1a = jnp.where(gt1a, e, a1a)
                gt1b = vb > m1b
                gt2b = vb > m2b
                m2b = jnp.where(gt1b, m1b, jnp.where(gt2b, vb, m2b))
                a2b = jnp.where(gt1b, a1b, jnp.where(gt2b, e, a2b))
                m1b = jnp.where(gt1b, vb, m1b)
                a1b = jnp.where(gt1b, e, a1b)
            return (m1a, m2a, a1a, a2a, m1b, m2b, a1b, a2b)

        neg = jnp.full((_L,), -1.0, jnp.float32)
        m1a, m2a, a1a, a2a, m1b, m2b, a1b, a2b = lax.fori_loop(
            0, _E // 8, _estep,
            (neg, neg, izeros, izeros, neg, neg, izeros, izeros))

        plsc.store_scatter(impv, [a1a, col0], m1a)
        plsc.store_scatter(impv, [a2a, col0], m2a)
        plsc.store_scatter(indv, [a1a, col0], ones)
        plsc.store_scatter(indv, [a2a, col0], ones)
        plsc.store_scatter(impv, [a1b, col1], m1b)
        plsc.store_scatter(impv, [a2b, col1], m2b)
        plsc.store_scatter(indv, [a1b, col1], ones)
        plsc.store_scatter(indv, [a2b, col1], ones)
        plsc.addupdate_scatter(cebuf, [a1a, lane], ones)
        plsc.addupdate_scatter(cebuf, [a1b, lane], ones)
        return c

    lax.fori_loop(0, _G // 2, _group_pair, 0)

    c1 = pltpu.async_copy(impv, imp_hbm.at[:, pl.ds(base, _TPW)], sem)
    c2 = pltpu.async_copy(indv, ind_hbm.at[:, pl.ds(base, _TPW)], sem)
    c1.wait()
    c2.wait()
    pltpu.sync_copy(cebuf, ce_hbm.at[wid])


_SC_MESH = plsc.VectorSubcoreMesh(
    core_axis_name="c", subcore_axis_name="s", num_cores=2, num_subcores=16)

_sc_route = pl.kernel(
    _route_body,
    out_type=[
        jax.ShapeDtypeStruct((_E, _NW * _TPW), jnp.float32),
        jax.ShapeDtypeStruct((_E, _NW * _TPW), jnp.float32),
        jax.ShapeDtypeStruct((_NW, _E, _L), jnp.float32),
    ],
    mesh=_SC_MESH,
    scratch_types=[
        pltpu.VMEM((_E, _TPW), jnp.float32),
        pltpu.VMEM((_E, _TPW), jnp.float32),
        pltpu.VMEM((_E, _TPW), jnp.float32),
        pltpu.VMEM((_E, _L), jnp.float32),
        pltpu.SemaphoreType.DMA,
    ],
    compiler_params=pltpu.CompilerParams(needs_layout_passes=False),
)


def kernel(x, W):
    B, S, D = x.shape
    n = B * S
    xf = x.reshape(n, D)
    p3, me_sum = _tc_gate(xf, W.T)
    imp3, ind3, ce_p = _sc_route(p3)
    imp = imp3.reshape(_E, B, S, 1)
    ind = ind3.reshape(_E, B, S, 1)
    ce = jnp.sum(ce_p, axis=(0, 2))
    loss = jnp.sum(me_sum[0] * ce) * (_E / float(n * n))
    return imp, ind, loss


# speedup vs baseline: 1.2434x; 1.0672x over previous
import jax
import jax.numpy as jnp
from jax import lax
from jax.experimental import pallas as pl
from jax.experimental.pallas import tpu as pltpu
from jax.experimental.pallas import tpu_sc as plsc

_E = 64
_NW = 32
_TPW = 256
_L = 16
_G = _TPW // _L


_TB = 1024


def _gate_body(x_ref, wt_ref, p_ref, me_ref, acc_ref):
    i = pl.program_id(0)

    @pl.when(i == 0)
    def _init():
        acc_ref[...] = jnp.zeros_like(acc_ref)

    logits = jnp.dot(x_ref[...], wt_ref[...],
                     preferred_element_type=jnp.float32)
    m = jnp.max(logits, axis=-1, keepdims=True)
    ex = jnp.exp(logits - m)
    s = jnp.sum(ex, axis=-1, keepdims=True)
    p = ex / s
    p_ref[...] = p.T
    acc_ref[...] += jnp.sum(p, axis=0, keepdims=True)

    @pl.when(i == pl.num_programs(0) - 1)
    def _fini():
        me_ref[...] = acc_ref[...]


def _tc_gate(xf, wt):
    n, d = xf.shape
    return pl.pallas_call(
        _gate_body,
        grid=(n // _TB,),
        in_specs=[
            pl.BlockSpec((_TB, d), lambda i: (i, 0)),
            pl.BlockSpec((d, _E), lambda i: (0, 0)),
        ],
        out_specs=[
            pl.BlockSpec((_E, _TB), lambda i: (0, i)),
            pl.BlockSpec((1, _E), lambda i: (0, 0)),
        ],
        out_shape=[
            jax.ShapeDtypeStruct((_E, n), jnp.float32),
            jax.ShapeDtypeStruct((1, _E), jnp.float32),
        ],
        scratch_shapes=[pltpu.VMEM((1, _E), jnp.float32)],
    )(xf, wt)


def _route_body(p_hbm, imp_hbm, ind_hbm, ce_hbm,
                lg, impv, indv, cebuf, sem):
    wid = lax.axis_index("s") * 2 + lax.axis_index("c")
    base = wid * _TPW
    cin = pltpu.async_copy(p_hbm.at[:, pl.ds(base, _TPW)], lg, sem)

    zeros = jnp.zeros((_L,), jnp.float32)

    def _zero_tiles(j, c):
        for g in range(_G):
            impv[j, pl.ds(g * _L, _L)] = zeros
            indv[j, pl.ds(g * _L, _L)] = zeros
        cebuf[j, :] = zeros
        return c

    lax.fori_loop(0, _E, _zero_tiles, 0)
    cin.wait()

    lane = lax.iota(jnp.int32, _L)
    ones = jnp.ones((_L,), jnp.float32)
    izeros = jnp.zeros((_L,), jnp.int32)

    def _group_pair(j, c):
        g0 = j * 2
        g1 = g0 + 1
        col0 = g0 * _L + lane
        col1 = g1 * _L + lane

        def _estep(eo, carry):
            m1a, m2a, a1a, a2a, m1b, m2b, a1b, a2b = carry
            for eu in range(8):
                e = eo * 8 + eu
                va = lg[e, pl.ds(g0 * _L, _L)]
                vb = lg[e, pl.ds(g1 * _L, _L)]
                gt1a = va > m1a
                gt2a = va > m2a
                m2a = jnp.where(gt1a, m1a, jnp.where(gt2a, va, m2a))
                a2a = jnp.where(gt1a, a1a, jnp.where(gt2a, e, a2a))
                m1a = jnp.where(gt1a, va, m1a)
                a1a = jnp.where(gt1a, e, a1a)
                gt1b = vb > m1b
                gt2b = vb > m2b
                m2b = jnp.where(gt1b, m1b, jnp.where(gt2b, vb, m2b))
                a2b = jnp.where(gt1b, a1b, jnp.where(gt2b, e, a2b))
                m1b = jnp.where(gt1b, vb, m1b)
                a1b = jnp.where(gt1b, e, a1b)
            return (m1a, m2a, a1a, a2a, m1b, m2b, a1b, a2b)

        neg = jnp.full((_L,), -1.0, jnp.float32)
        m1a, m2a, a1a, a2a, m1b, m2b, a1b, a2b = lax.fori_loop(
            0, _E // 8, _estep,
            (neg, neg, izeros, izeros, neg, neg, izeros, izeros))

        plsc.store_scatter(impv, [a1a, col0], m1a)
        plsc.store_scatter(impv, [a2a, col0], m2a)
        plsc.store_scatter(indv, [a1a, col0], ones)
        plsc.store_scatter(indv, [a2a, col0], ones)
        plsc.store_scatter(impv, [a1b, col1], m1b)
        plsc.store_scatter(impv, [a2b, col1], m2b)
        plsc.store_scatter(indv, [a1b, col1], ones)
        plsc.store_scatter(indv, [a2b, col1], ones)
        plsc.addupdate_scatter(cebuf, [a1a, lane], ones)
        plsc.addupdate_scatter(cebuf, [a1b, lane], ones)
        return c

    lax.fori_loop(0, _G // 2, _group_pair, 0)

    c1 = pltpu.async_copy(impv, imp_hbm.at[:, pl.ds(base, _TPW)], sem)
    c2 = pltpu.async_copy(indv, ind_hbm.at[:, pl.ds(base, _TPW)], sem)
    c1.wait()
    c2.wait()
    pltpu.sync_copy(cebuf, ce_hbm.at[wid])


_SC_MESH = plsc.VectorSubcoreMesh(
    core_axis_name="c", subcore_axis_name="s", num_cores=2, num_subcores=16)

_sc_route = pl.kernel(
    _route_body,
    out_type=[
        jax.ShapeDtypeStruct((_E, _NW * _TPW), jnp.float32),
        jax.ShapeDtypeStruct((_E, _NW * _TPW), jnp.float32),
        jax.ShapeDtypeStruct((_NW, _E, _L), jnp.float32),
    ],
    mesh=_SC_MESH,
    scratch_types=[
        pltpu.VMEM((_E, _TPW), jnp.float32),
        pltpu.VMEM((_E, _TPW), jnp.float32),
        pltpu.VMEM((_E, _TPW), jnp.float32),
        pltpu.VMEM((_E, _L), jnp.float32),
        pltpu.SemaphoreType.DMA,
    ],
    compiler_params=pltpu.CompilerParams(needs_layout_passes=False),
)


def kernel(x, W):
    B, S, D = x.shape
    n = B * S
    xf = x.reshape(n, D)
    p3, me_sum = _tc_gate(xf, W.T)
    imp3, ind3, ce_p = _sc_route(p3)
    imp = imp3.reshape(_E, B, S, 1)
    ind = ind3.reshape(_E, B, S, 1)
    ce = jnp.sum(ce_p, axis=(0, 2))
    loss = jnp.sum(me_sum[0] * ce) * (_E / float(n * n))
    return imp, ind, loss
